# Initial kernel scaffold; baseline (speedup 1.0000x reference)
import jax
import jax.numpy as jnp
from jax.experimental import pallas as pl


def kernel(graph_node, edge_index, W_P_w, W_P_b, W_pos, qT0, kT0, vT0, gamma0, beta0, qT1, kT1, vT1, gamma1, beta1, inv_w, inv_b):
    raise NotImplementedError("write your pallas kernel here")


# speedup vs baseline: 1.1588x; 1.1588x over previous
import functools

import jax
import jax.numpy as jnp
from jax import lax
from jax.experimental import pallas as pl
from jax.experimental.pallas import tpu as pltpu
from jax.experimental.pallas import tpu_sc as plsc

N = 10000
E = 320000
D = 128
H = 4
DH = D // H
L = 16

NC = 2
NS = 16
NW = NC * NS
CH = 128
NCHUNK = E // CH
TITER = -(-NCHUNK // NW)

CH2 = 512
NCHUNK2 = E // CH2
TITER2 = -(-NCHUNK2 // NW)

N2 = 10240
NF = N2 * H

ROWB = 1000
GRID = N // ROWB

_mesh = plsc.VectorSubcoreMesh(core_axis_name="c", subcore_axis_name="s")
_scp = pltpu.CompilerParams(needs_layout_passes=False)


def _dotT(x, w):
    return lax.dot_general(x, w, (((1,), (1,)), ((), ())),
                           preferred_element_type=jnp.float32)


def _dot(x, w):
    return lax.dot_general(x, w, (((1,), (0,)), ((), ())),
                           preferred_element_type=jnp.float32)


def _ln(res, gamma, beta):
    mu = jnp.mean(res, axis=-1, keepdims=True)
    var = jnp.mean(jnp.square(res - mu), axis=-1, keepdims=True)
    return (res - mu) / jnp.sqrt(var + 1e-06) * gamma + beta


def _tc_pre(x, wp, bp, pos, qt, kt, vt):
    def body(x_ref, wp_ref, bp_ref, pos_ref, qt_ref, kt_ref, vt_ref,
             emb_ref, q_ref, k_ref, v_ref):
        z = _dotT(x_ref[...], wp_ref[...]) + bp_ref[...] + pos_ref[...]
        emb_ref[...] = z
        q_ref[...] = _dot(z, qt_ref[...])
        k_ref[...] = _dot(z, kt_ref[...])
        v_ref[...] = _dot(z, vt_ref[...])

    full = pl.BlockSpec((D, D), lambda i: (0, 0))
    vec = pl.BlockSpec((1, D), lambda i: (0, 0))
    rowb = pl.BlockSpec((ROWB, D), lambda i: (i, 0))
    shp = jax.ShapeDtypeStruct((N, D), jnp.float32)
    return pl.pallas_call(
        body, grid=(GRID,),
        in_specs=[rowb, full, vec, vec, full, full, full],
        out_specs=[rowb, rowb, rowb, rowb],
        out_shape=[shp, shp, shp, shp],
    )(x, wp, bp.reshape(1, D), pos, qt, kt, vt)


def _combine(oa, ob, dsum, bmat, emb):
    den = jnp.sum(dsum, axis=0)
    den128 = _dot(den, bmat) + 1e-08
    return (oa + ob) / den128 + emb


def _tc_mid(oa, ob, dall, bmat, emb, gamma, beta, qt, kt, vt):
    def body(oa_ref, ob_ref, d_ref, bm_ref, emb_ref, g_ref, b_ref,
             qt_ref, kt_ref, vt_ref, emb2_ref, q_ref, k_ref, v_ref):
        res = _combine(oa_ref[...], ob_ref[...], d_ref[...],
                       bm_ref[...], emb_ref[...])
        z = _ln(res, g_ref[...], b_ref[...])
        emb2_ref[...] = z
        q_ref[...] = _dot(z, qt_ref[...])
        k_ref[...] = _dot(z, kt_ref[...])
        v_ref[...] = _dot(z, vt_ref[...])

    full = pl.BlockSpec((D, D), lambda i: (0, 0))
    vec = pl.BlockSpec((1, D), lambda i: (0, 0))
    rowb = pl.BlockSpec((ROWB, D), lambda i: (i, 0))
    dsp = pl.BlockSpec((NW, ROWB, H), lambda i: (0, i, 0))
    bms = pl.BlockSpec((H, D), lambda i: (0, 0))
    shp = jax.ShapeDtypeStruct((N, D), jnp.float32)
    return pl.pallas_call(
        body, grid=(GRID,),
        in_specs=[rowb, rowb, dsp, bms, rowb, vec, vec, full, full, full],
        out_specs=[rowb, rowb, rowb, rowb],
        out_shape=[shp, shp, shp, shp],
    )(oa, ob, dall, bmat, emb, gamma.reshape(1, D), beta.reshape(1, D),
      qt, kt, vt)


def _tc_post(oa, ob, dall, bmat, emb, gamma, beta, wi, bi):
    def body(oa_ref, ob_ref, d_ref, bm_ref, emb_ref, g_ref, b_ref,
             wi_ref, bi_ref, out_ref):
        res = _combine(oa_ref[...], ob_ref[...], d_ref[...],
                       bm_ref[...], emb_ref[...])
        z = _ln(res, g_ref[...], b_ref[...])
        out_ref[...] = _dotT(z, wi_ref[...]) + bi_ref[...]

    full = pl.BlockSpec((D, D), lambda i: (0, 0))
    vec = pl.BlockSpec((1, D), lambda i: (0, 0))
    rowb = pl.BlockSpec((ROWB, D), lambda i: (i, 0))
    dsp = pl.BlockSpec((NW, ROWB, H), lambda i: (0, i, 0))
    bms = pl.BlockSpec((H, D), lambda i: (0, 0))
    return pl.pallas_call(
        body, grid=(GRID,),
        in_specs=[rowb, rowb, dsp, bms, rowb, vec, vec, full, vec],
        out_specs=rowb,
        out_shape=jax.ShapeDtypeStruct((N, D), jnp.float32),
    )(oa, ob, dall, bmat, emb, gamma.reshape(1, D), beta.reshape(1, D),
      wi, bi.reshape(1, D))


def _splat(val):
    return jnp.full((L,), val, jnp.int32)


def _sc_layer(q, k, v, rows, cols, znd):
    @functools.partial(
        pl.kernel,
        out_type=[jax.ShapeDtypeStruct((N, D), jnp.float32),
                  jax.ShapeDtypeStruct((N, D), jnp.float32),
                  jax.ShapeDtypeStruct((H, E), jnp.float32)],
        mesh=_mesh,
        compiler_params=_scp,
        scratch_types=[
            pltpu.VMEM((1, CH), jnp.int32),
            pltpu.VMEM((1, CH), jnp.int32),
            pltpu.VMEM((CH, D), jnp.float32),
            pltpu.VMEM((CH, D), jnp.float32),
            pltpu.VMEM((CH, D), jnp.float32),
            pltpu.VMEM((H, CH), jnp.float32),
            pltpu.VMEM_SHARED((N, D), jnp.float32),
            pltpu.SemaphoreType.DMA,
        ],
    )
    def sck(q_hbm, k_hbm, v_hbm, rows_hbm, cols_hbm, znd_hbm,
            oa_hbm, ob_hbm, ea_hbm,
            ridx, cidx, qbuf, kbuf, vbuf, attbuf, numacc, sem):
        c = lax.axis_index("c")
        s = lax.axis_index("s")
        wid = c * NS + s

        @pl.when(s == 0)
        def _():
            pltpu.sync_copy(znd_hbm, numacc)
        plsc.subcore_barrier()

        def chunk_body(i, carry):
            cid = wid + i * NW

            @pl.when(cid < NCHUNK)
            def _():
                base = cid * CH
                pltpu.sync_copy(rows_hbm.at[pl.ds(base, CH)], ridx.at[0])
                pltpu.sync_copy(cols_hbm.at[pl.ds(base, CH)], cidx.at[0])
                cps = [
                    pltpu.async_copy(q_hbm.at[ridx.at[0]], qbuf, sem),
                    pltpu.async_copy(k_hbm.at[cidx.at[0]], kbuf, sem),
                    pltpu.async_copy(v_hbm.at[cidx.at[0]], vbuf, sem),
                ]
                for cp in cps:
                    cp.wait()

                def scores_body(g, carry2):
                    eidx = g * L + lax.iota(jnp.int32, L)
                    for h in range(H):
                        def d_body(d, acc):
                            dd = _splat(h * DH) + d
                            qv = plsc.load_gather(qbuf, [eidx, dd])
                            kv = plsc.load_gather(kbuf, [eidx, dd])
                            return acc + qv * kv
                        acc = lax.fori_loop(0, DH, d_body,
                                            jnp.zeros((L,), jnp.float32),
                                            unroll=8)
                        att = jnp.exp(jnp.clip(acc, -10.0, 10.0))
                        attbuf[h, pl.ds(g * L, L)] = att
                    return carry2
                lax.fori_loop(0, CH // L, scores_body, 0)

                for h in range(H):
                    pltpu.sync_copy(attbuf.at[h],
                                    ea_hbm.at[h].at[pl.ds(base, CH)])

                def scale_body(g, carry2):
                    eidx = g * L + lax.iota(jnp.int32, L)
                    for h in range(H):
                        attv = attbuf[h, pl.ds(g * L, L)]

                        def d_body(d, carry3):
                            dd = _splat(h * DH) + d
                            col = plsc.load_gather(vbuf, [eidx, dd])
                            plsc.store_scatter(qbuf, [eidx, dd], attv * col)
                            return carry3
                        lax.fori_loop(0, DH, d_body, 0, unroll=8)
                    return carry2
                lax.fori_loop(0, CH // L, scale_body, 0)

                pltpu.sync_copy(qbuf, numacc.at[ridx.at[0]], add=True)
            return carry
        lax.fori_loop(0, TITER, chunk_body, 0)

        plsc.subcore_barrier()

        @pl.when(s == 0)
        def _():
            @pl.when(c == 0)
            def _():
                pltpu.sync_copy(numacc, oa_hbm)

            @pl.when(c == 1)
            def _():
                pltpu.sync_copy(numacc, ob_hbm)

    return sck(q, k, v, rows, cols, znd)


def _sc_den(ea, rows, znf):
    @functools.partial(
        pl.kernel,
        out_type=jax.ShapeDtypeStruct((NW, NF), jnp.float32),
        mesh=_mesh,
        compiler_params=_scp,
        scratch_types=[
            pltpu.VMEM((CH2,), jnp.int32),
            pltpu.VMEM((H, CH2), jnp.float32),
            pltpu.VMEM((NF,), jnp.float32),
        ],
    )
    def denk(ea_hbm, rows_hbm, znf_hbm, den_hbm, ridx1d, eabuf, denpriv):
        c = lax.axis_index("c")
        s = lax.axis_index("s")
        wid = c * NS + s

        pltpu.sync_copy(znf_hbm, denpriv)

        def chunk_body(i, carry):
            cid = wid + i * NW

            @pl.when(cid < NCHUNK2)
            def _():
                base = cid * CH2
                pltpu.sync_copy(rows_hbm.at[pl.ds(base, CH2)], ridx1d)
                for h in range(H):
                    pltpu.sync_copy(ea_hbm.at[h].at[pl.ds(base, CH2)],
                                    eabuf.at[h])

                def group_body(g, carry2):
                    nodes = ridx1d[pl.ds(g * L, L)]
                    slot = nodes * H
                    for h in range(H):
                        att = eabuf[h, pl.ds(g * L, L)]
                        plsc.addupdate_scatter(denpriv, [slot + h], att)
                    return carry2
                lax.fori_loop(0, CH2 // L, group_body, 0)
            return carry
        lax.fori_loop(0, TITER2, chunk_body, 0)

        pltpu.sync_copy(denpriv, den_hbm.at[wid])

    return denk(ea, rows, znf)


def kernel(graph_node, edge_index, W_P_w, W_P_b, W_pos, qT0, kT0, vT0,
           gamma0, beta0, qT1, kT1, vT1, gamma1, beta1, inv_w, inv_b):
    rows = edge_index[0]
    cols = edge_index[1]
    znd = jnp.zeros((N, D), jnp.float32)
    znf = jnp.zeros((NF,), jnp.float32)
    bmat = jnp.repeat(jnp.eye(H, dtype=jnp.float32), DH, axis=1)

    emb0, q0, k0, v0 = _tc_pre(graph_node, W_P_w, W_P_b, W_pos, qT0, kT0, vT0)
    oa0, ob0, ea0 = _sc_layer(q0, k0, v0, rows, cols, znd)
    dall0 = _sc_den(ea0, rows, znf).reshape(NW, N2, H)

    emb1, q1, k1, v1 = _tc_mid(oa0, ob0, dall0, bmat, emb0,
                               gamma0, beta0, qT1, kT1, vT1)
    oa1, ob1, ea1 = _sc_layer(q1, k1, v1, rows, cols, znd)
    dall1 = _sc_den(ea1, rows, znf).reshape(NW, N2, H)

    return _tc_post(oa1, ob1, dall1, bmat, emb1,
                    gamma1, beta1, inv_w, inv_b)
